# baseline (speedup 1.0000x reference)
import functools
import math

import jax
import jax.numpy as jnp
from jax import lax
from jax.experimental import pallas as pl
from jax.experimental.pallas import tpu as pltpu
from jax.experimental.pallas import tpu_sc as plsc

S = 200
B = 1024
D = 64
SCALE = 8.0
N = 1000000

NC = 2
NS = 16
NW = NC * NS
BGRP = 4
SGRP = NW // BGRP
S_PER = S // SGRP
B_PER = B // BGRP
LANES = 16

K0_COLS = 2048


def _make_pe(d_model, max_len):
    position = jnp.arange(0, max_len, dtype=jnp.float32)[:, None]
    div_term = jnp.exp(
        jnp.arange(0, d_model, 2, dtype=jnp.float32) * (-math.log(10000.0) / d_model)
    )
    pe = jnp.zeros((max_len, d_model), dtype=jnp.float32)
    pe = pe.at[:, 0::2].set(jnp.sin(position * div_term))
    pe = pe.at[:, 1::2].set(jnp.cos(position * div_term))
    return pe


def _transpose_block(a_ref, o_ref):
    at = jnp.transpose(a_ref[...], (1, 0))
    o_ref[...] = jnp.concatenate(
        [at, jnp.zeros((K0_COLS, D), jnp.float32)], axis=1
    )


def _widen_table(tab_t):
    grid = (N + K0_COLS - 1) // K0_COLS
    return pl.pallas_call(
        _transpose_block,
        grid=(grid,),
        in_specs=[pl.BlockSpec((D, K0_COLS), lambda k: (0, k))],
        out_specs=pl.BlockSpec((K0_COLS, 2 * D), lambda k: (k, 0)),
        out_shape=jax.ShapeDtypeStruct((N, 2 * D), jnp.float32),
    )(tab_t)


@functools.partial(
    pl.kernel,
    mesh=plsc.VectorSubcoreMesh(core_axis_name="c", subcore_axis_name="s"),
    compiler_params=pltpu.CompilerParams(
        use_tc_tiling_on_sc=False, needs_layout_passes=False
    ),
    out_type=jax.ShapeDtypeStruct((S, D, B), jnp.float32),
    scratch_types=[
        pltpu.VMEM((2, 128), jnp.int32),
        pltpu.VMEM((B_PER, 2 * D), jnp.float32),
        pltpu.VMEM((D, B_PER), jnp.float32),
        pltpu.VMEM((2 * D,), jnp.float32),
        pltpu.SemaphoreType.DMA,
    ],
)
def _emb_kernel(xt_hbm, pe_hbm, tab_hbm, out_hbm, idx_v, g_v, o_v, pe_v, sem):
    wid = lax.axis_index("s") * NC + lax.axis_index("c")
    sgrp = wid // BGRP
    bq = wid % BGRP
    s_lo = sgrp * S_PER
    r0 = bq * 2
    b0 = bq * B_PER

    lanes_iota = lax.iota(jnp.int32, LANES)

    def body(i, carry):
        s = s_lo + i
        pltpu.sync_copy(xt_hbm.at[s, pl.ds(r0, 2)], idx_v)
        pltpu.sync_copy(pe_hbm.at[s], pe_v)
        cp0 = pltpu.async_copy(tab_hbm.at[idx_v.at[0]], g_v.at[pl.ds(0, 128)], sem)
        cp1 = pltpu.async_copy(tab_hbm.at[idx_v.at[1]], g_v.at[pl.ds(128, 128)], sem)
        cp0.wait()
        cp1.wait()

        def col(j, ccarry):
            jv = jnp.zeros((LANES,), jnp.int32) + j
            pe_s = plsc.load_gather(pe_v, [jv]) * (1.0 / SCALE)
            for t in range(B_PER // LANES):
                bvec = LANES * t + lanes_iota
                vals = plsc.load_gather(g_v, [bvec, jv])
                o_v[j, pl.ds(LANES * t, LANES)] = (vals + pe_s) * SCALE
            return ccarry

        lax.fori_loop(0, D, col, 0)
        pltpu.sync_copy(o_v, out_hbm.at[s, pl.ds(0, D), pl.ds(b0, B_PER)])
        return carry

    lax.fori_loop(0, S_PER, body, 0)


def kernel(x, emb_table):
    xt = jnp.reshape(jnp.transpose(x.astype(jnp.int32), (1, 0)), (S, 8, 128))
    tab_t = jnp.transpose(emb_table, (1, 0))
    tab_wide = _widen_table(tab_t)
    pe = _make_pe(D, S)
    pe_wide = jnp.concatenate([pe, pe], axis=1)
    out = _emb_kernel(xt, pe_wide, tab_wide)
    return jnp.transpose(out, (0, 2, 1))
